# baseline (speedup 1.0000x reference)
import functools

import jax
import jax.numpy as jnp
from jax import lax
from jax.experimental import pallas as pl
from jax.experimental.pallas import tpu as pltpu
from jax.experimental.pallas import tpu_sc as plsc

N = 10000
NP = 10240
E = 320000
F = 16
D_IN = 128
CH = 80
NC = 2
NS = 16
NW = NC * NS
TPW = 125
RPS = NP // NS
NBUF = 5
NQ = 8


def _sc_degree(dst2d):
    mesh = plsc.VectorSubcoreMesh(core_axis_name="c", subcore_axis_name="s", num_cores=NC, num_subcores=NS)

    @functools.partial(
        pl.kernel,
        out_type=jax.ShapeDtypeStruct((NC, 1, NP), jnp.float32),
        mesh=mesh,
        scratch_types=[
            pltpu.VMEM((TPW, CH), jnp.int32),
            pltpu.VMEM((CH,), jnp.float32),
            pltpu.VMEM((RPS,), jnp.float32),
            pltpu.VMEM_SHARED((NP,), jnp.float32),
            pltpu.SemaphoreType.DMA,
        ],
    )
    def k(dst_hbm, out_hbm, idx_v, ones_v, buf_v, deg_sh, dsem):
        c = lax.axis_index("c")
        s = lax.axis_index("s")
        w = c * NS + s
        one = jnp.ones((16,), jnp.float32)
        zero = jnp.zeros((16,), jnp.float32)

        def fill_ones(i, _):
            ones_v[pl.ds(i * 16, 16)] = one
            return 0

        lax.fori_loop(0, CH // 16, fill_ones, 0)

        def fill_zero(i, _):
            buf_v[pl.ds(i * 16, 16)] = zero
            return 0

        lax.fori_loop(0, RPS // 16, fill_zero, 0)
        pltpu.sync_copy(buf_v, deg_sh.at[pl.ds(s * RPS, RPS)])
        plsc.subcore_barrier()

        pltpu.sync_copy(dst_hbm.at[w], idx_v)

        def body(j, _):
            pltpu.async_copy(ones_v, deg_sh.at[idx_v.at[j]], dsem, add=True)

            @pl.when(j >= NQ)
            def _():
                pltpu.make_async_copy(ones_v, deg_sh.at[idx_v.at[0]], dsem).wait()

            return 0

        lax.fori_loop(0, TPW, body, 0)
        for _ in range(NQ):
            pltpu.make_async_copy(ones_v, deg_sh.at[idx_v.at[0]], dsem).wait()
        plsc.subcore_barrier()

        pltpu.sync_copy(deg_sh.at[pl.ds(s * RPS, RPS)], buf_v)
        pltpu.sync_copy(buf_v, out_hbm.at[c, 0, pl.ds(s * RPS, RPS)])

    return k(dst2d)


def _sc_aggregate(g, src2d, dst2d):
    mesh = plsc.VectorSubcoreMesh(core_axis_name="c", subcore_axis_name="s", num_cores=NC, num_subcores=NS)

    @functools.partial(
        pl.kernel,
        out_type=jax.ShapeDtypeStruct((NC, NP, F), jnp.float32),
        mesh=mesh,
        compiler_params=pltpu.CompilerParams(use_tc_tiling_on_sc=False),
        scratch_types=[
            pltpu.VMEM((TPW, CH), jnp.int32),
            pltpu.VMEM((TPW, CH), jnp.int32),
            pltpu.VMEM((NBUF, CH, F), jnp.float32),
            pltpu.VMEM((RPS, F), jnp.float32),
            pltpu.VMEM_SHARED((NP, F), jnp.float32),
            pltpu.SemaphoreType.DMA((NBUF,)),
            pltpu.SemaphoreType.DMA((NBUF,)),
        ],
    )
    def k(g_hbm, src_hbm, dst_hbm, out_hbm, src_v, dst_v, rows_v, buf_v, acc_sh, gsem, ssem):
        c = lax.axis_index("c")
        s = lax.axis_index("s")
        w = c * NS + s
        zero = jnp.zeros((F,), jnp.float32)

        def fill_zero(i, _):
            buf_v[i, :] = zero
            return 0

        lax.fori_loop(0, RPS, fill_zero, 0)
        pltpu.sync_copy(buf_v, acc_sh.at[pl.ds(s * RPS, RPS)])
        plsc.subcore_barrier()

        pltpu.sync_copy(src_hbm.at[w], src_v)
        pltpu.sync_copy(dst_hbm.at[w], dst_v)

        for b in range(NBUF):
            pltpu.async_copy(g_hbm.at[src_v.at[b]], rows_v.at[b], gsem.at[b])

        nit = TPW // NBUF

        def body(i, _):
            j = i * NBUF
            for b in range(NBUF):
                pltpu.make_async_copy(
                    g_hbm.at[src_v.at[j + b]], rows_v.at[b], gsem.at[b]
                ).wait()
                pltpu.async_copy(
                    rows_v.at[b], acc_sh.at[dst_v.at[j + b]], ssem.at[b], add=True
                )
            for b in range(NBUF):
                pltpu.make_async_copy(
                    rows_v.at[b], acc_sh.at[dst_v.at[j + b]], ssem.at[b]
                ).wait()

                @pl.when(i < nit - 1)
                def _():
                    pltpu.async_copy(
                        g_hbm.at[src_v.at[j + NBUF + b]], rows_v.at[b], gsem.at[b]
                    )

            return 0

        lax.fori_loop(0, nit, body, 0)
        plsc.subcore_barrier()

        pltpu.sync_copy(acc_sh.at[pl.ds(s * RPS, RPS)], buf_v)
        pltpu.sync_copy(buf_v, out_hbm.at[c, pl.ds(s * RPS, RPS)])

    return k(g, src2d, dst2d)


def _tc_layer1(x, W1, degp_t):
    def body(x_ref, w_ref, d_ref, o_ref):
        deg = 1.0 + d_ref[:, 0:1] + d_ref[:, 1:2]
        dinv = lax.rsqrt(deg)
        h = jnp.dot(x_ref[...], w_ref[...], preferred_element_type=jnp.float32)
        o_ref[0:N] = h * dinv[0:N]

    return pl.pallas_call(
        body, out_shape=jax.ShapeDtypeStruct((NP, F), jnp.float32)
    )(x, W1, degp_t)


def _tc_layer2(aggp, g1, degp_t, W2, b1r):
    def body(a_ref, g_ref, d_ref, w_ref, b_ref, o_ref):
        deg = 1.0 + d_ref[:, 0:1] + d_ref[:, 1:2]
        dinv = lax.rsqrt(deg)
        a = (a_ref[0] + a_ref[1] + g_ref[...]) * dinv + b_ref[...]
        h = jnp.maximum(a, 0.0)
        o_ref[...] = jnp.dot(h, w_ref[...], preferred_element_type=jnp.float32) * dinv

    return pl.pallas_call(
        body, out_shape=jax.ShapeDtypeStruct((NP, F), jnp.float32)
    )(aggp, g1, degp_t, W2, b1r)


def _tc_final(aggp, g2, degp_t, b2r):
    def body(a_ref, g_ref, d_ref, b_ref, o_ref):
        deg = 1.0 + d_ref[0:N, 0:1] + d_ref[0:N, 1:2]
        dinv = lax.rsqrt(deg)
        a = (a_ref[0, 0:N] + a_ref[1, 0:N] + g_ref[0:N]) * dinv + b_ref[...]
        m = jnp.max(a, axis=1, keepdims=True)
        e = jnp.exp(a - m)
        ssum = jnp.sum(e, axis=1, keepdims=True)
        o_ref[...] = (a - m) - jnp.log(ssum)

    return pl.pallas_call(
        body, out_shape=jax.ShapeDtypeStruct((N, F), jnp.float32)
    )(aggp, g2, degp_t, b2r)


def kernel(x, edge_index, W1, b1, W2, b2):
    src = edge_index[0].astype(jnp.int32).reshape(NW, TPW, CH)
    dst = edge_index[1].astype(jnp.int32).reshape(NW, TPW, CH)

    degp = _sc_degree(dst)
    degp_t = degp.reshape(NC, NP).T
    g1 = _tc_layer1(x, W1, degp_t)
    p1 = _sc_aggregate(g1, src, dst)
    g2 = _tc_layer2(p1, g1, degp_t, W2, b1.reshape(1, F))
    p2 = _sc_aggregate(g2, src, dst)
    out = _tc_final(p2, g2, degp_t, b2.reshape(1, F))
    return out


# speedup vs baseline: 1.4324x; 1.4324x over previous
import functools

import jax
import jax.numpy as jnp
from jax import lax
from jax.experimental import pallas as pl
from jax.experimental.pallas import tpu as pltpu
from jax.experimental.pallas import tpu_sc as plsc

N = 10000
NP = 10240
E = 320000
F = 16
D_IN = 128
CH = 80
NC = 2
NS = 16
NW = NC * NS
TPW = 125
RPS = NP // NS
NBUF = 5
NQ = 8
PR = NP // 8
PN = N // 8


def _sc_degree(ei):
    mesh = plsc.VectorSubcoreMesh(core_axis_name="c", subcore_axis_name="s", num_cores=NC, num_subcores=NS)

    @functools.partial(
        pl.kernel,
        out_type=jax.ShapeDtypeStruct((NC, NP, F), jnp.float32),
        mesh=mesh,
        compiler_params=pltpu.CompilerParams(use_tc_tiling_on_sc=False),
        scratch_types=[
            pltpu.VMEM((TPW, CH), jnp.int32),
            pltpu.VMEM((CH,), jnp.float32),
            pltpu.VMEM((RPS,), jnp.float32),
            pltpu.VMEM((RPS, F), jnp.float32),
            pltpu.VMEM_SHARED((NP,), jnp.float32),
            pltpu.SemaphoreType.DMA,
        ],
    )
    def k(ei_hbm, out_hbm, idx_v, ones_v, buf_v, buf16_v, deg_sh, dsem):
        c = lax.axis_index("c")
        s = lax.axis_index("s")
        w = c * NS + s
        one = jnp.ones((16,), jnp.float32)
        zero = jnp.zeros((16,), jnp.float32)

        def fill_ones(i, _):
            ones_v[pl.ds(i * 16, 16)] = one
            return 0

        lax.fori_loop(0, CH // 16, fill_ones, 0)

        def fill_zero(i, _):
            buf_v[pl.ds(i * 16, 16)] = zero
            return 0

        lax.fori_loop(0, RPS // 16, fill_zero, 0)
        pltpu.sync_copy(buf_v, deg_sh.at[pl.ds(s * RPS, RPS)])
        plsc.subcore_barrier()

        pltpu.sync_copy(ei_hbm.at[1, w], idx_v)

        def body(j, _):
            pltpu.async_copy(ones_v, deg_sh.at[idx_v.at[j]], dsem, add=True)

            @pl.when(j >= NQ)
            def _():
                pltpu.make_async_copy(ones_v, deg_sh.at[idx_v.at[0]], dsem).wait()

            return 0

        lax.fori_loop(0, TPW, body, 0)
        for _ in range(NQ):
            pltpu.make_async_copy(ones_v, deg_sh.at[idx_v.at[0]], dsem).wait()
        plsc.subcore_barrier()

        pltpu.sync_copy(deg_sh.at[pl.ds(s * RPS, RPS)], buf_v)

        def splat(i, _):
            vec = buf_v[pl.ds(i * 16, 16)]
            for kk in range(16):
                buf16_v[i * 16 + kk, :] = one * vec[kk]
            return 0

        lax.fori_loop(0, RPS // 16, splat, 0)
        pltpu.sync_copy(buf16_v, out_hbm.at[c, pl.ds(s * RPS, RPS)])

    return k(ei)


def _sc_aggregate(g, ei):
    mesh = plsc.VectorSubcoreMesh(core_axis_name="c", subcore_axis_name="s", num_cores=NC, num_subcores=NS)

    @functools.partial(
        pl.kernel,
        out_type=jax.ShapeDtypeStruct((NC, NP, F), jnp.float32),
        mesh=mesh,
        compiler_params=pltpu.CompilerParams(use_tc_tiling_on_sc=False),
        scratch_types=[
            pltpu.VMEM((TPW, CH), jnp.int32),
            pltpu.VMEM((TPW, CH), jnp.int32),
            pltpu.VMEM((NBUF, CH, F), jnp.float32),
            pltpu.VMEM((RPS, F), jnp.float32),
            pltpu.VMEM_SHARED((NP, F), jnp.float32),
            pltpu.SemaphoreType.DMA((NBUF,)),
            pltpu.SemaphoreType.DMA((NBUF,)),
        ],
    )
    def k(g_hbm, ei_hbm, out_hbm, src_v, dst_v, rows_v, buf_v, acc_sh, gsem, ssem):
        c = lax.axis_index("c")
        s = lax.axis_index("s")
        w = c * NS + s
        zero = jnp.zeros((F,), jnp.float32)

        def fill_zero(i, _):
            buf_v[i, :] = zero
            return 0

        lax.fori_loop(0, RPS, fill_zero, 0)
        pltpu.sync_copy(buf_v, acc_sh.at[pl.ds(s * RPS, RPS)])
        plsc.subcore_barrier()

        pltpu.sync_copy(ei_hbm.at[0, w], src_v)
        pltpu.sync_copy(ei_hbm.at[1, w], dst_v)

        for b in range(NBUF):
            pltpu.async_copy(g_hbm.at[src_v.at[b]], rows_v.at[b], gsem.at[b])

        nit = TPW // NBUF

        def body(i, _):
            j = i * NBUF
            for b in range(NBUF):
                pltpu.make_async_copy(
                    g_hbm.at[src_v.at[j + b]], rows_v.at[b], gsem.at[b]
                ).wait()
                pltpu.async_copy(
                    rows_v.at[b], acc_sh.at[dst_v.at[j + b]], ssem.at[b], add=True
                )
            for b in range(NBUF):
                pltpu.make_async_copy(
                    rows_v.at[b], acc_sh.at[dst_v.at[j + b]], ssem.at[b]
                ).wait()

                @pl.when(i < nit - 1)
                def _():
                    pltpu.async_copy(
                        g_hbm.at[src_v.at[j + NBUF + b]], rows_v.at[b], gsem.at[b]
                    )

            return 0

        lax.fori_loop(0, nit, body, 0)
        plsc.subcore_barrier()

        pltpu.sync_copy(acc_sh.at[pl.ds(s * RPS, RPS)], buf_v)
        pltpu.sync_copy(buf_v, out_hbm.at[c, pl.ds(s * RPS, RPS)])

    return k(g, ei)


def _dinv_packed(d_ref):
    return lax.rsqrt(1.0 + d_ref[0, 0:PN] + d_ref[1, 0:PN])


def _tc_layer1(x, W1, degB):
    def body(x_ref, w_ref, d_ref, o_ref):
        dinv = _dinv_packed(d_ref)
        w128 = jnp.tile(w_ref[...], (1, 8))
        ht = jnp.dot(x_ref[...], w128, preferred_element_type=jnp.float32)
        rowmod = lax.broadcasted_iota(jnp.int32, (N, 128), 0) % 8
        grp = lax.broadcasted_iota(jnp.int32, (N, 128), 1) // F
        hm = jnp.where(rowmod == grp, ht, 0.0).reshape(PN, 8, 128)
        p = (
            hm[:, 0, :] + hm[:, 1, :] + hm[:, 2, :] + hm[:, 3, :]
            + hm[:, 4, :] + hm[:, 5, :] + hm[:, 6, :] + hm[:, 7, :]
        )
        o_ref[0:PN] = p * dinv

    return pl.pallas_call(
        body, out_shape=jax.ShapeDtypeStruct((PR, 128), jnp.float32)
    )(x, W1, degB)


def _block_diag_mask(val128):
    gi = lax.broadcasted_iota(jnp.int32, (128, 128), 0) // F
    gj = lax.broadcasted_iota(jnp.int32, (128, 128), 1) // F
    return jnp.where(gi == gj, val128, 0.0)


def _tc_layer2(aggB, g1p, degB, W2, b1r):
    def body(a_ref, g_ref, d_ref, w_ref, b_ref, o_ref):
        dinv = _dinv_packed(d_ref)
        b1p = jnp.tile(b_ref[...], (1, 8))
        a = (a_ref[0, 0:PN] + a_ref[1, 0:PN] + g_ref[0:PN]) * dinv + b1p
        h = jnp.maximum(a, 0.0)
        w2b = _block_diag_mask(jnp.tile(w_ref[...], (8, 8)))
        p2 = jnp.dot(h, w2b, preferred_element_type=jnp.float32)
        o_ref[0:PN] = p2 * dinv

    return pl.pallas_call(
        body, out_shape=jax.ShapeDtypeStruct((PR, 128), jnp.float32)
    )(aggB, g1p, degB, W2, b1r)


def _tc_final(aggB, g2p, degB, b2r):
    def body(a_ref, g_ref, d_ref, b_ref, o_ref):
        dinv = _dinv_packed(d_ref)
        b2p = jnp.tile(b_ref[...], (1, 8))
        a = (a_ref[0, 0:PN] + a_ref[1, 0:PN] + g_ref[0:PN]) * dinv + b2p
        e = jnp.exp(a)
        ones_b = _block_diag_mask(jnp.ones((128, 128), jnp.float32))
        ssum = jnp.dot(e, ones_b, preferred_element_type=jnp.float32,
                       precision=lax.Precision.HIGHEST)
        outp = a - jnp.log(ssum)
        t = jnp.broadcast_to(outp[:, None, :], (PN, 8, 128)).reshape(N, 128)
        rowmod = lax.broadcasted_iota(jnp.int32, (N, 128), 0) % 8
        grp = lax.broadcasted_iota(jnp.int32, (N, 128), 1) // F
        tm = jnp.where(rowmod == grp, t, 0.0)
        li = lax.broadcasted_iota(jnp.int32, (128, F), 0) % F
        lj = lax.broadcasted_iota(jnp.int32, (128, F), 1)
        sel = jnp.where(li == lj, 1.0, 0.0)
        o_ref[...] = jnp.dot(tm, sel, preferred_element_type=jnp.float32,
                             precision=lax.Precision.HIGHEST)

    return pl.pallas_call(
        body, out_shape=jax.ShapeDtypeStruct((N, F), jnp.float32)
    )(aggB, g2p, degB, b2r)


def kernel(x, edge_index, W1, b1, W2, b2):
    ei = edge_index.astype(jnp.int32).reshape(2, NW, TPW, CH)

    degp = _sc_degree(ei)
    degB = degp.reshape(NC, PR, 128)
    g1p = _tc_layer1(x, W1, degB)
    p1 = _sc_aggregate(g1p.reshape(NP, F), ei)
    g2p = _tc_layer2(p1.reshape(NC, PR, 128), g1p, degB, W2, b1.reshape(1, F))
    p2 = _sc_aggregate(g2p.reshape(NP, F), ei)
    out = _tc_final(p2.reshape(NC, PR, 128), g2p, degB, b2.reshape(1, F))
    return out


# speedup vs baseline: 1.5607x; 1.0896x over previous
import functools

import jax
import jax.numpy as jnp
from jax import lax
from jax.experimental import pallas as pl
from jax.experimental.pallas import tpu as pltpu
from jax.experimental.pallas import tpu_sc as plsc

N = 10000
NP = 10240
E = 320000
F = 16
D_IN = 128
CH = 80
NC = 2
NS = 16
NW = NC * NS
TPW = 125
RPS = NP // NS
NBUF = 5
NQ = 8
PR = NP // 8
PN = N // 8


def _sc_degree(ei):
    mesh = plsc.VectorSubcoreMesh(core_axis_name="c", subcore_axis_name="s", num_cores=NC, num_subcores=NS)

    @functools.partial(
        pl.kernel,
        out_type=jax.ShapeDtypeStruct((NC, NP, F), jnp.float32),
        mesh=mesh,
        compiler_params=pltpu.CompilerParams(use_tc_tiling_on_sc=False),
        scratch_types=[
            pltpu.VMEM((TPW, CH), jnp.int32),
            pltpu.VMEM((CH,), jnp.float32),
            pltpu.VMEM((RPS,), jnp.float32),
            pltpu.VMEM((RPS, F), jnp.float32),
            pltpu.VMEM_SHARED((NP,), jnp.float32),
            pltpu.SemaphoreType.DMA,
        ],
    )
    def k(ei_hbm, out_hbm, idx_v, ones_v, buf_v, buf16_v, deg_sh, dsem):
        c = lax.axis_index("c")
        s = lax.axis_index("s")
        w = c * NS + s
        one = jnp.ones((16,), jnp.float32)
        zero = jnp.zeros((16,), jnp.float32)

        def fill_ones(i, _):
            ones_v[pl.ds(i * 16, 16)] = one
            return 0

        lax.fori_loop(0, CH // 16, fill_ones, 0)

        def fill_zero(i, _):
            buf_v[pl.ds(i * 16, 16)] = zero
            return 0

        lax.fori_loop(0, RPS // 16, fill_zero, 0)
        pltpu.sync_copy(buf_v, deg_sh.at[pl.ds(s * RPS, RPS)])
        plsc.subcore_barrier()

        pltpu.sync_copy(ei_hbm.at[1, w], idx_v)

        def body(j, _):
            pltpu.async_copy(ones_v, deg_sh.at[idx_v.at[j]], dsem, add=True)

            @pl.when(j >= NQ)
            def _():
                pltpu.make_async_copy(ones_v, deg_sh.at[idx_v.at[0]], dsem).wait()

            return 0

        lax.fori_loop(0, TPW, body, 0)
        for _ in range(NQ):
            pltpu.make_async_copy(ones_v, deg_sh.at[idx_v.at[0]], dsem).wait()
        plsc.subcore_barrier()

        pltpu.sync_copy(deg_sh.at[pl.ds(s * RPS, RPS)], buf_v)

        def splat(i, _):
            vec = buf_v[pl.ds(i * 16, 16)]
            for kk in range(16):
                buf16_v[i * 16 + kk, :] = one * vec[kk]
            return 0

        lax.fori_loop(0, RPS // 16, splat, 0)
        pltpu.sync_copy(buf16_v, out_hbm.at[c, pl.ds(s * RPS, RPS)])

    return k(ei)


def _sc_aggregate(g, ei):
    mesh = plsc.VectorSubcoreMesh(core_axis_name="c", subcore_axis_name="s", num_cores=NC, num_subcores=NS)

    @functools.partial(
        pl.kernel,
        out_type=jax.ShapeDtypeStruct((NC, NP, F), jnp.float32),
        mesh=mesh,
        compiler_params=pltpu.CompilerParams(use_tc_tiling_on_sc=False),
        scratch_types=[
            pltpu.VMEM((TPW, CH), jnp.int32),
            pltpu.VMEM((TPW, CH), jnp.int32),
            pltpu.VMEM((NBUF, CH, F), jnp.float32),
            pltpu.VMEM((RPS, F), jnp.float32),
            pltpu.VMEM_SHARED((NP, F), jnp.float32),
            pltpu.VMEM_SHARED((NP, F), jnp.float32),
            pltpu.SemaphoreType.DMA((NBUF,)),
            pltpu.SemaphoreType.DMA((NBUF,)),
        ],
    )
    def k(g_hbm, ei_hbm, out_hbm, src_v, dst_v, rows_v, buf_v, acc_sh, g_sh, gsem, ssem):
        c = lax.axis_index("c")
        s = lax.axis_index("s")
        w = c * NS + s
        zero = jnp.zeros((F,), jnp.float32)

        def fill_zero(i, _):
            buf_v[i, :] = zero
            return 0

        lax.fori_loop(0, RPS, fill_zero, 0)
        pltpu.sync_copy(buf_v, acc_sh.at[pl.ds(s * RPS, RPS)])
        pltpu.sync_copy(g_hbm.at[pl.ds(s * RPS, RPS)], g_sh.at[pl.ds(s * RPS, RPS)])
        plsc.subcore_barrier()

        pltpu.sync_copy(ei_hbm.at[0, w], src_v)
        pltpu.sync_copy(ei_hbm.at[1, w], dst_v)

        for b in range(NBUF):
            pltpu.async_copy(g_sh.at[src_v.at[b]], rows_v.at[b], gsem.at[b])

        nit = TPW // NBUF

        def body(i, _):
            j = i * NBUF
            for b in range(NBUF):
                pltpu.make_async_copy(
                    g_sh.at[src_v.at[j + b]], rows_v.at[b], gsem.at[b]
                ).wait()
                pltpu.async_copy(
                    rows_v.at[b], acc_sh.at[dst_v.at[j + b]], ssem.at[b], add=True
                )
            for b in range(NBUF):
                pltpu.make_async_copy(
                    rows_v.at[b], acc_sh.at[dst_v.at[j + b]], ssem.at[b]
                ).wait()

                @pl.when(i < nit - 1)
                def _():
                    pltpu.async_copy(
                        g_sh.at[src_v.at[j + NBUF + b]], rows_v.at[b], gsem.at[b]
                    )

            return 0

        lax.fori_loop(0, nit, body, 0)
        plsc.subcore_barrier()

        pltpu.sync_copy(acc_sh.at[pl.ds(s * RPS, RPS)], buf_v)
        pltpu.sync_copy(buf_v, out_hbm.at[c, pl.ds(s * RPS, RPS)])

    return k(g, ei)


def _dinv_packed(d_ref):
    return lax.rsqrt(1.0 + d_ref[0, 0:PN] + d_ref[1, 0:PN])


def _tc_layer1(x, W1, degB):
    def body(x_ref, w_ref, d_ref, o_ref):
        dinv = _dinv_packed(d_ref)
        w128 = jnp.tile(w_ref[...], (1, 8))
        ht = jnp.dot(x_ref[...], w128, preferred_element_type=jnp.float32)
        rowmod = lax.broadcasted_iota(jnp.int32, (N, 128), 0) % 8
        grp = lax.broadcasted_iota(jnp.int32, (N, 128), 1) // F
        hm = jnp.where(rowmod == grp, ht, 0.0).reshape(PN, 8, 128)
        p = (
            hm[:, 0, :] + hm[:, 1, :] + hm[:, 2, :] + hm[:, 3, :]
            + hm[:, 4, :] + hm[:, 5, :] + hm[:, 6, :] + hm[:, 7, :]
        )
        o_ref[0:PN] = p * dinv

    return pl.pallas_call(
        body, out_shape=jax.ShapeDtypeStruct((PR, 128), jnp.float32)
    )(x, W1, degB)


def _block_diag_mask(val128):
    gi = lax.broadcasted_iota(jnp.int32, (128, 128), 0) // F
    gj = lax.broadcasted_iota(jnp.int32, (128, 128), 1) // F
    return jnp.where(gi == gj, val128, 0.0)


def _tc_layer2(aggB, g1p, degB, W2, b1r):
    def body(a_ref, g_ref, d_ref, w_ref, b_ref, o_ref):
        dinv = _dinv_packed(d_ref)
        b1p = jnp.tile(b_ref[...], (1, 8))
        a = (a_ref[0, 0:PN] + a_ref[1, 0:PN] + g_ref[0:PN]) * dinv + b1p
        h = jnp.maximum(a, 0.0)
        w2b = _block_diag_mask(jnp.tile(w_ref[...], (8, 8)))
        p2 = jnp.dot(h, w2b, preferred_element_type=jnp.float32)
        o_ref[0:PN] = p2 * dinv

    return pl.pallas_call(
        body, out_shape=jax.ShapeDtypeStruct((PR, 128), jnp.float32)
    )(aggB, g1p, degB, W2, b1r)


def _tc_final(aggB, g2p, degB, b2r):
    def body(a_ref, g_ref, d_ref, b_ref, o_ref):
        dinv = _dinv_packed(d_ref)
        b2p = jnp.tile(b_ref[...], (1, 8))
        a = (a_ref[0, 0:PN] + a_ref[1, 0:PN] + g_ref[0:PN]) * dinv + b2p
        e = jnp.exp(a)
        ones_b = _block_diag_mask(jnp.ones((128, 128), jnp.float32))
        ssum = jnp.dot(e, ones_b, preferred_element_type=jnp.float32,
                       precision=lax.Precision.HIGHEST)
        outp = a - jnp.log(ssum)
        t = jnp.broadcast_to(outp[:, None, :], (PN, 8, 128)).reshape(N, 128)
        rowmod = lax.broadcasted_iota(jnp.int32, (N, 128), 0) % 8
        grp = lax.broadcasted_iota(jnp.int32, (N, 128), 1) // F
        tm = jnp.where(rowmod == grp, t, 0.0)
        li = lax.broadcasted_iota(jnp.int32, (128, F), 0) % F
        lj = lax.broadcasted_iota(jnp.int32, (128, F), 1)
        sel = jnp.where(li == lj, 1.0, 0.0)
        o_ref[...] = jnp.dot(tm, sel, preferred_element_type=jnp.float32,
                             precision=lax.Precision.HIGHEST)

    return pl.pallas_call(
        body, out_shape=jax.ShapeDtypeStruct((N, F), jnp.float32)
    )(aggB, g2p, degB, b2r)


def kernel(x, edge_index, W1, b1, W2, b2):
    ei = edge_index.astype(jnp.int32).reshape(2, NW, TPW, CH)

    degp = _sc_degree(ei)
    degB = degp.reshape(NC, PR, 128)
    g1p = _tc_layer1(x, W1, degB)
    p1 = _sc_aggregate(g1p.reshape(NP, F), ei)
    g2p = _tc_layer2(p1.reshape(NC, PR, 128), g1p, degB, W2, b1.reshape(1, F))
    p2 = _sc_aggregate(g2p.reshape(NP, F), ei)
    out = _tc_final(p2.reshape(NC, PR, 128), g2p, degB, b2.reshape(1, F))
    return out


# speedup vs baseline: 1.6252x; 1.0413x over previous
import functools

import jax
import jax.numpy as jnp
from jax import lax
from jax.experimental import pallas as pl
from jax.experimental.pallas import tpu as pltpu
from jax.experimental.pallas import tpu_sc as plsc

N = 10000
NP = 10240
E = 320000
F = 16
D_IN = 128
CH = 80
NC = 2
NS = 16
NW = NC * NS
TPW = 125
RPS = NP // NS
NBUF = 5
NQ = 8
PR = NP // 8
PN = N // 8


def _sc_degree(ei):
    mesh = plsc.VectorSubcoreMesh(core_axis_name="c", subcore_axis_name="s", num_cores=NC, num_subcores=NS)

    @functools.partial(
        pl.kernel,
        out_type=jax.ShapeDtypeStruct((NC, NP, F), jnp.float32),
        mesh=mesh,
        compiler_params=pltpu.CompilerParams(use_tc_tiling_on_sc=False),
        scratch_types=[
            pltpu.VMEM((TPW, CH), jnp.int32),
            pltpu.VMEM((CH,), jnp.float32),
            pltpu.VMEM((RPS,), jnp.float32),
            pltpu.VMEM((RPS, F), jnp.float32),
            pltpu.VMEM_SHARED((NP,), jnp.float32),
            pltpu.SemaphoreType.DMA,
        ],
    )
    def k(ei_hbm, out_hbm, idx_v, ones_v, buf_v, buf16_v, deg_sh, dsem):
        c = lax.axis_index("c")
        s = lax.axis_index("s")
        w = c * NS + s
        one = jnp.ones((16,), jnp.float32)
        zero = jnp.zeros((16,), jnp.float32)

        def fill_ones(i, _):
            ones_v[pl.ds(i * 16, 16)] = one
            return 0

        lax.fori_loop(0, CH // 16, fill_ones, 0)

        def fill_zero(i, _):
            buf_v[pl.ds(i * 16, 16)] = zero
            return 0

        lax.fori_loop(0, RPS // 16, fill_zero, 0)
        pltpu.sync_copy(buf_v, deg_sh.at[pl.ds(s * RPS, RPS)])
        plsc.subcore_barrier()

        pltpu.sync_copy(ei_hbm.at[1, w], idx_v)

        def body(j, _):
            pltpu.async_copy(ones_v, deg_sh.at[idx_v.at[j]], dsem, add=True)

            @pl.when(j >= NQ)
            def _():
                pltpu.make_async_copy(ones_v, deg_sh.at[idx_v.at[0]], dsem).wait()

            return 0

        lax.fori_loop(0, TPW, body, 0)
        for _ in range(NQ):
            pltpu.make_async_copy(ones_v, deg_sh.at[idx_v.at[0]], dsem).wait()
        plsc.subcore_barrier()

        pltpu.sync_copy(deg_sh.at[pl.ds(s * RPS, RPS)], buf_v)

        def splat(i, _):
            vec = buf_v[pl.ds(i * 16, 16)]
            for kk in range(16):
                buf16_v[i * 16 + kk, :] = one * vec[kk]
            return 0

        lax.fori_loop(0, RPS // 16, splat, 0)
        pltpu.sync_copy(buf16_v, out_hbm.at[c, pl.ds(s * RPS, RPS)])

    return k(ei)


def _sc_aggregate(g, ei):
    mesh = plsc.VectorSubcoreMesh(core_axis_name="c", subcore_axis_name="s", num_cores=NC, num_subcores=NS)

    @functools.partial(
        pl.kernel,
        out_type=jax.ShapeDtypeStruct((NC, NP, F), jnp.float32),
        mesh=mesh,
        compiler_params=pltpu.CompilerParams(use_tc_tiling_on_sc=False),
        scratch_types=[
            pltpu.VMEM((TPW, CH), jnp.int32),
            pltpu.VMEM((TPW, CH), jnp.int32),
            pltpu.VMEM((NBUF, CH, F), jnp.float32),
            pltpu.VMEM((RPS, F), jnp.float32),
            pltpu.VMEM_SHARED((NP, F), jnp.float32),
            pltpu.VMEM_SHARED((NP, F), jnp.float32),
            pltpu.SemaphoreType.DMA((NBUF,)),
            pltpu.SemaphoreType.DMA((NBUF,)),
        ],
    )
    def k(g_hbm, ei_hbm, out_hbm, src_v, dst_v, rows_v, buf_v, acc_sh, g_sh, gsem, ssem):
        c = lax.axis_index("c")
        s = lax.axis_index("s")
        w = c * NS + s
        zero = jnp.zeros((F,), jnp.float32)

        def fill_zero(i, _):
            buf_v[i, :] = zero
            return 0

        lax.fori_loop(0, RPS, fill_zero, 0)
        pltpu.sync_copy(buf_v, acc_sh.at[pl.ds(s * RPS, RPS)])
        pltpu.sync_copy(g_hbm.at[pl.ds(s * RPS, RPS)], g_sh.at[pl.ds(s * RPS, RPS)])
        plsc.subcore_barrier()

        pltpu.sync_copy(ei_hbm.at[0, w], src_v)
        pltpu.sync_copy(ei_hbm.at[1, w], dst_v)

        for b in range(NBUF):
            pltpu.async_copy(g_sh.at[src_v.at[b]], rows_v.at[b], gsem.at[b])

        nit = TPW // NBUF

        def body(i, _):
            j = i * NBUF
            for b in range(NBUF):
                pltpu.make_async_copy(
                    g_sh.at[src_v.at[j + b]], rows_v.at[b], gsem.at[b]
                ).wait()
                pltpu.async_copy(
                    rows_v.at[b], acc_sh.at[dst_v.at[j + b]], ssem.at[b], add=True
                )
            for b in range(NBUF):
                pltpu.make_async_copy(
                    rows_v.at[b], acc_sh.at[dst_v.at[j + b]], ssem.at[b]
                ).wait()

                @pl.when(i < nit - 1)
                def _():
                    pltpu.async_copy(
                        g_sh.at[src_v.at[j + NBUF + b]], rows_v.at[b], gsem.at[b]
                    )

            return 0

        lax.fori_loop(0, nit, body, 0)
        plsc.subcore_barrier()

        pltpu.sync_copy(acc_sh.at[pl.ds(s * RPS, RPS)], buf_v)
        pltpu.sync_copy(buf_v, out_hbm.at[c, pl.ds(s * RPS, RPS)])

    return k(g, ei)


def _dinv_packed(d_ref):
    return lax.rsqrt(1.0 + d_ref[0, 0:PN] + d_ref[1, 0:PN])


def _tc_layer1(x, W1, degB):
    def body(x_ref, w_ref, d_ref, o_ref):
        dinv = _dinv_packed(d_ref)
        w128 = jnp.tile(w_ref[...], (1, 8))
        ht = jnp.dot(x_ref[...], w128, preferred_element_type=jnp.float32)
        rowmod = lax.broadcasted_iota(jnp.int32, (N, 128), 0) % 8
        grp = lax.broadcasted_iota(jnp.int32, (N, 128), 1) // F
        hm = jnp.where(rowmod == grp, ht, 0.0).reshape(PN, 8, 128)
        p = (
            hm[:, 0, :] + hm[:, 1, :] + hm[:, 2, :] + hm[:, 3, :]
            + hm[:, 4, :] + hm[:, 5, :] + hm[:, 6, :] + hm[:, 7, :]
        )
        o_ref[0:PN] = p * dinv

    return pl.pallas_call(
        body, out_shape=jax.ShapeDtypeStruct((PR, 128), jnp.float32)
    )(x, W1, degB)


def _block_diag_mask(val128):
    gi = lax.broadcasted_iota(jnp.int32, (128, 128), 0) // F
    gj = lax.broadcasted_iota(jnp.int32, (128, 128), 1) // F
    return jnp.where(gi == gj, val128, 0.0)


def _tc_layer2(aggB, g1p, degB, W2, b1r):
    def body(a_ref, g_ref, d_ref, w_ref, b_ref, o_ref):
        dinv = _dinv_packed(d_ref)
        b1p = jnp.tile(b_ref[...], (1, 8))
        a = (a_ref[0, 0:PN] + a_ref[1, 0:PN] + g_ref[0:PN]) * dinv + b1p
        h = jnp.maximum(a, 0.0)
        w2b = _block_diag_mask(jnp.tile(w_ref[...], (8, 8)))
        p2 = jnp.dot(h, w2b, preferred_element_type=jnp.float32)
        o_ref[0:PN] = p2 * dinv

    return pl.pallas_call(
        body, out_shape=jax.ShapeDtypeStruct((PR, 128), jnp.float32)
    )(aggB, g1p, degB, W2, b1r)


def _tc_final(aggB, g2p, degB, b2r):
    def body(a_ref, g_ref, d_ref, b_ref, o_ref):
        dinv = _dinv_packed(d_ref)
        b2p = jnp.tile(b_ref[...], (1, 8))
        a = (a_ref[0, 0:PN] + a_ref[1, 0:PN] + g_ref[0:PN]) * dinv + b2p
        e = jnp.exp(a)
        ones_b = _block_diag_mask(jnp.ones((128, 128), jnp.float32))
        ssum = jnp.dot(e, ones_b, preferred_element_type=jnp.float32,
                       precision=lax.Precision.HIGHEST)
        o_ref[0:PN] = a - jnp.log(ssum)

    return pl.pallas_call(
        body, out_shape=jax.ShapeDtypeStruct((PR, 128), jnp.float32)
    )(aggB, g2p, degB, b2r)


def kernel(x, edge_index, W1, b1, W2, b2):
    ei = edge_index.astype(jnp.int32).reshape(2, NW, TPW, CH)

    degp = _sc_degree(ei)
    degB = degp.reshape(NC, PR, 128)
    g1p = _tc_layer1(x, W1, degB)
    p1 = _sc_aggregate(g1p.reshape(NP, F), ei)
    g2p = _tc_layer2(p1.reshape(NC, PR, 128), g1p, degB, W2, b1.reshape(1, F))
    p2 = _sc_aggregate(g2p.reshape(NP, F), ei)
    outp = _tc_final(p2.reshape(NC, PR, 128), g2p, degB, b2.reshape(1, F))
    return outp.reshape(NP, F)[:N]


# speedup vs baseline: 1.6431x; 1.0110x over previous
import functools

import jax
import jax.numpy as jnp
from jax import lax
from jax.experimental import pallas as pl
from jax.experimental.pallas import tpu as pltpu
from jax.experimental.pallas import tpu_sc as plsc

N = 10000
NP = 10240
E = 320000
F = 16
D_IN = 128
CH = 80
NC = 2
NS = 16
NW = NC * NS
TPW = 125
RPS = NP // NS
NBUF = 5
NQ = 8
PR = NP // 8
PN = N // 8


def _sc_degree(ei):
    mesh = plsc.VectorSubcoreMesh(core_axis_name="c", subcore_axis_name="s", num_cores=NC, num_subcores=NS)

    @functools.partial(
        pl.kernel,
        out_type=jax.ShapeDtypeStruct((NC, NP, F), jnp.float32),
        mesh=mesh,
        compiler_params=pltpu.CompilerParams(use_tc_tiling_on_sc=False),
        scratch_types=[
            pltpu.VMEM((TPW, CH), jnp.int32),
            pltpu.VMEM((CH,), jnp.float32),
            pltpu.VMEM((RPS,), jnp.float32),
            pltpu.VMEM((RPS, F), jnp.float32),
            pltpu.VMEM_SHARED((NP,), jnp.float32),
            pltpu.SemaphoreType.DMA,
        ],
    )
    def k(ei_hbm, out_hbm, idx_v, ones_v, buf_v, buf16_v, deg_sh, dsem):
        c = lax.axis_index("c")
        s = lax.axis_index("s")
        w = c * NS + s
        one = jnp.ones((16,), jnp.float32)
        zero = jnp.zeros((16,), jnp.float32)

        def fill_ones(i, _):
            ones_v[pl.ds(i * 16, 16)] = one
            return 0

        lax.fori_loop(0, CH // 16, fill_ones, 0)

        def fill_zero(i, _):
            buf_v[pl.ds(i * 16, 16)] = zero
            return 0

        lax.fori_loop(0, RPS // 16, fill_zero, 0)
        pltpu.sync_copy(buf_v, deg_sh.at[pl.ds(s * RPS, RPS)])
        plsc.subcore_barrier()

        pltpu.sync_copy(ei_hbm.at[1, w], idx_v)

        def body(j, _):
            pltpu.async_copy(ones_v, deg_sh.at[idx_v.at[j]], dsem, add=True)

            @pl.when(j >= NQ)
            def _():
                pltpu.make_async_copy(ones_v, deg_sh.at[idx_v.at[0]], dsem).wait()

            return 0

        lax.fori_loop(0, TPW, body, 0)
        for _ in range(NQ):
            pltpu.make_async_copy(ones_v, deg_sh.at[idx_v.at[0]], dsem).wait()
        plsc.subcore_barrier()

        pltpu.sync_copy(deg_sh.at[pl.ds(s * RPS, RPS)], buf_v)

        def splat(i, _):
            vec = buf_v[pl.ds(i * 16, 16)]
            for kk in range(16):
                buf16_v[i * 16 + kk, :] = one * vec[kk]
            return 0

        lax.fori_loop(0, RPS // 16, splat, 0)
        pltpu.sync_copy(buf16_v, out_hbm.at[c, pl.ds(s * RPS, RPS)])

    return k(ei)


def _sc_aggregate(g, ei):
    mesh = plsc.VectorSubcoreMesh(core_axis_name="c", subcore_axis_name="s", num_cores=NC, num_subcores=NS)

    @functools.partial(
        pl.kernel,
        out_type=jax.ShapeDtypeStruct((NC, NP, F), jnp.float32),
        mesh=mesh,
        compiler_params=pltpu.CompilerParams(use_tc_tiling_on_sc=False),
        scratch_types=[
            pltpu.VMEM((TPW, CH), jnp.int32),
            pltpu.VMEM((TPW, CH), jnp.int32),
            pltpu.VMEM((NBUF, CH, F), jnp.float32),
            pltpu.VMEM((RPS, F), jnp.float32),
            pltpu.VMEM_SHARED((NP, F), jnp.float32),
            pltpu.VMEM_SHARED((NP, F), jnp.float32),
            pltpu.SemaphoreType.DMA((NBUF,)),
            pltpu.SemaphoreType.DMA((NBUF,)),
        ],
    )
    def k(g_hbm, ei_hbm, out_hbm, src_v, dst_v, rows_v, buf_v, acc_sh, g_sh, gsem, ssem):
        c = lax.axis_index("c")
        s = lax.axis_index("s")
        w = c * NS + s
        zero = jnp.zeros((F,), jnp.float32)

        def fill_zero(i, _):
            buf_v[i, :] = zero
            return 0

        lax.fori_loop(0, RPS, fill_zero, 0)
        pltpu.sync_copy(buf_v, acc_sh.at[pl.ds(s * RPS, RPS)])
        pltpu.sync_copy(g_hbm.at[pl.ds(s * RPS, RPS)], g_sh.at[pl.ds(s * RPS, RPS)])
        plsc.subcore_barrier()

        pltpu.sync_copy(ei_hbm.at[0, w], src_v)
        pltpu.sync_copy(ei_hbm.at[1, w], dst_v)

        for b in range(NBUF):
            pltpu.async_copy(g_sh.at[src_v.at[b]], rows_v.at[b], gsem.at[b])

        nit = TPW // NBUF

        def body(i, _):
            j = i * NBUF
            for b in range(NBUF):
                pltpu.make_async_copy(
                    g_sh.at[src_v.at[j + b]], rows_v.at[b], gsem.at[b]
                ).wait()
                pltpu.async_copy(
                    rows_v.at[b], acc_sh.at[dst_v.at[j + b]], ssem.at[b], add=True
                )
            for b in range(NBUF):
                pltpu.make_async_copy(
                    rows_v.at[b], acc_sh.at[dst_v.at[j + b]], ssem.at[b]
                ).wait()

                @pl.when(i < nit - 1)
                def _():
                    pltpu.async_copy(
                        g_sh.at[src_v.at[j + NBUF + b]], rows_v.at[b], gsem.at[b]
                    )

            return 0

        lax.fori_loop(0, nit, body, 0)
        plsc.subcore_barrier()

        pltpu.sync_copy(acc_sh.at[pl.ds(s * RPS, RPS)], buf_v)
        pltpu.sync_copy(buf_v, out_hbm.at[c, pl.ds(s * RPS, RPS)])

    return k(g, ei)


def _dinv_packed(d_ref):
    return lax.rsqrt(1.0 + d_ref[0, 0:PN] + d_ref[1, 0:PN])


def _tc_layer1(x, W1, degB):
    def body(x_ref, w_ref, d_ref, o_ref):
        dinv = _dinv_packed(d_ref)
        h = jnp.dot(x_ref[...], w_ref[...], preferred_element_type=jnp.float32)
        y = h.reshape(PN, 8, F)
        p = jnp.concatenate([y[:, k, :] for k in range(8)], axis=1)
        o_ref[0:PN] = p * dinv

    return pl.pallas_call(
        body, out_shape=jax.ShapeDtypeStruct((PR, 128), jnp.float32)
    )(x, W1, degB)


def _block_diag_mask(val128):
    gi = lax.broadcasted_iota(jnp.int32, (128, 128), 0) // F
    gj = lax.broadcasted_iota(jnp.int32, (128, 128), 1) // F
    return jnp.where(gi == gj, val128, 0.0)


def _tc_layer2(aggB, g1p, degB, W2, b1r):
    def body(a_ref, g_ref, d_ref, w_ref, b_ref, o_ref):
        dinv = _dinv_packed(d_ref)
        b1p = jnp.tile(b_ref[...], (1, 8))
        a = (a_ref[0, 0:PN] + a_ref[1, 0:PN] + g_ref[0:PN]) * dinv + b1p
        h = jnp.maximum(a, 0.0)
        w2b = _block_diag_mask(jnp.tile(w_ref[...], (8, 8)))
        p2 = jnp.dot(h, w2b, preferred_element_type=jnp.float32)
        o_ref[0:PN] = p2 * dinv

    return pl.pallas_call(
        body, out_shape=jax.ShapeDtypeStruct((PR, 128), jnp.float32)
    )(aggB, g1p, degB, W2, b1r)


def _tc_final(aggB, g2p, degB, b2r):
    def body(a_ref, g_ref, d_ref, b_ref, o_ref):
        dinv = _dinv_packed(d_ref)
        b2p = jnp.tile(b_ref[...], (1, 8))
        a = (a_ref[0, 0:PN] + a_ref[1, 0:PN] + g_ref[0:PN]) * dinv + b2p
        e = jnp.exp(a)
        ones_b = _block_diag_mask(jnp.ones((128, 128), jnp.float32))
        ssum = jnp.dot(e, ones_b, preferred_element_type=jnp.float32,
                       precision=lax.Precision.HIGHEST)
        o_ref[0:PN] = a - jnp.log(ssum)

    return pl.pallas_call(
        body, out_shape=jax.ShapeDtypeStruct((PR, 128), jnp.float32)
    )(aggB, g2p, degB, b2r)


def kernel(x, edge_index, W1, b1, W2, b2):
    ei = edge_index.astype(jnp.int32).reshape(2, NW, TPW, CH)

    degp = _sc_degree(ei)
    degB = degp.reshape(NC, PR, 128)
    g1p = _tc_layer1(x, W1, degB)
    p1 = _sc_aggregate(g1p.reshape(NP, F), ei)
    g2p = _tc_layer2(p1.reshape(NC, PR, 128), g1p, degB, W2, b1.reshape(1, F))
    p2 = _sc_aggregate(g2p.reshape(NP, F), ei)
    outp = _tc_final(p2.reshape(NC, PR, 128), g2p, degB, b2.reshape(1, F))
    return outp.reshape(NP, F)[:N]
